# Initial kernel scaffold; baseline (speedup 1.0000x reference)
import jax
import jax.numpy as jnp
from jax.experimental import pallas as pl


def kernel(x, edge_index, edge_attr, W_same, b_same, W_diff, b_diff, W_G, W_last, b_last, u):
    raise NotImplementedError("write your pallas kernel here")


# speedup vs baseline: 3.9364x; 3.9364x over previous
import functools

import jax
import jax.numpy as jnp
from jax import lax
from jax.experimental import pallas as pl
from jax.experimental.pallas import tpu as pltpu
from jax.experimental.pallas import tpu_sc as plsc

_PREC = lax.Precision.DEFAULT

_NUM_CORES = 2
_NUM_SUBCORES = 16
_LANES = 16
_CHUNK = 40
_NBUF = 2


def _ssp(v):
    return jax.nn.softplus(v) - jnp.log(2.0)


def _row_matmul(a, w):
    return lax.dot_general(a, w, (((1,), (1,)), ((), ())), precision=_PREC)


def _prelude_body(x_ref, wd_ref, bd_ref, ws_ref, bs_ref, h_ref, a2_ref):
    xa = _ssp(x_ref[...])
    h_ref[...] = _ssp(_row_matmul(xa, wd_ref[...]) + bd_ref[...])
    a2_ref[...] = 0.5 * _ssp(_row_matmul(xa, ws_ref[...]) + bs_ref[...])


def _g_body(ea_ref, wg_ref, g_ref):
    g_ref[...] = _row_matmul(ea_ref[...], wg_ref[...])


def _tail_body(p0_ref, p1_ref, x_ref, wl_ref, bl_ref, u_ref, out_ref, m_ref):
    m = p0_ref[...] + p1_ref[...]
    v = _row_matmul(_ssp(m), wl_ref[...]) + bl_ref[...]
    m_ref[...] = m
    out_ref[...] = v + x_ref[...] * u_ref[...]


_NI = 6


def _make_sc_aggregate(N, F, E):
    n_workers = _NUM_CORES * _NUM_SUBCORES
    epw = E // n_workers
    n_chunks = epw // _CHUNK
    assert epw % _CHUNK == 0
    main_lo, main_hi = 4, n_chunks - 6
    assert (main_hi - main_lo) % 6 == 0 and main_hi > main_lo
    slab_a = (N // _NUM_SUBCORES) // 8 * 8
    slab_last = N - slab_a * (_NUM_SUBCORES - 1)
    assert slab_a % 8 == 0 and slab_last % 8 == 0 and slab_last > 0

    mesh = plsc.VectorSubcoreMesh(core_axis_name="c", subcore_axis_name="s")

    @functools.partial(
        pl.kernel,
        mesh=mesh,
        out_type=jax.ShapeDtypeStruct((_NUM_CORES, N, F), jnp.float32),
        scratch_types=[
            pltpu.VMEM((_NI, 2, _CHUNK), jnp.int32),
            pltpu.VMEM((_NBUF, _CHUNK, F), jnp.float32),
            pltpu.VMEM((_NBUF, _CHUNK, F), jnp.float32),
            pltpu.VMEM((_NBUF, _CHUNK, F), jnp.float32),
            pltpu.VMEM_SHARED((N, F), jnp.float32),
            pltpu.SemaphoreType.DMA((_NI,)),
            pltpu.SemaphoreType.DMA((_NBUF,)),
            pltpu.SemaphoreType.DMA((_NBUF,)),
        ],
    )
    def sc_aggregate(h_hbm, g_hbm, idx_hbm, a2_hbm, out_hbm,
                     islot, rows, gbuf, mbuf, acc, isem, gsem, ssem):
        c = lax.axis_index("c")
        s = lax.axis_index("s")
        wid = c * _NUM_SUBCORES + s
        ebase = wid * epw

        @pl.when(s < _NUM_SUBCORES - 1)
        def _():
            slab = pl.ds(s * slab_a, slab_a)
            pltpu.sync_copy(a2_hbm.at[slab, :], acc.at[slab, :])

        @pl.when(s == _NUM_SUBCORES - 1)
        def _():
            slab = pl.ds(s * slab_a, slab_last)
            pltpu.sync_copy(a2_hbm.at[slab, :], acc.at[slab, :])

        plsc.subcore_barrier()

        def issue_idx(j, ib):
            pltpu.async_copy(idx_hbm.at[wid, j], islot.at[ib], isem.at[ib])

        def wait_idx(j, ib):
            pltpu.make_async_copy(idx_hbm.at[wid, j], islot.at[ib],
                                  isem.at[ib]).wait()

        def issue_fetch(j, b, ib):
            pltpu.async_copy(h_hbm.at[islot.at[ib, 0]], rows.at[b], gsem.at[b])
            pltpu.async_copy(g_hbm.at[pl.ds(ebase + j * _CHUNK, _CHUNK), :],
                             gbuf.at[b], gsem.at[b])

        def wait_fetch(j, b, ib):
            pltpu.make_async_copy(h_hbm.at[islot.at[ib, 0]], rows.at[b],
                                  gsem.at[b]).wait()
            pltpu.make_async_copy(g_hbm.at[pl.ds(ebase + j * _CHUNK, _CHUNK), :],
                                  gbuf.at[b], gsem.at[b]).wait()

        def issue_scatter(b, ib):
            pltpu.async_copy(mbuf.at[b], acc.at[islot.at[ib, 1]], ssem.at[b],
                             add=True)

        def wait_scatter(b, ib):
            pltpu.make_async_copy(mbuf.at[b], acc.at[islot.at[ib, 1]],
                                  ssem.at[b]).wait()

        def compute(b):
            @pl.loop(0, _CHUNK)
            def _(i):
                for f in range(0, F, _LANES):
                    sl = pl.ds(f, _LANES)
                    mbuf[b, i, sl] = rows[b, i, sl] * gbuf[b, i, sl]

        def step(j, b, ib_cur, ib_wait, ib_new,
                 do_ws=True, do_idx=True, do_fetch=True):
            if do_ws:
                wait_scatter(b, ib_cur)
            wait_fetch(j, b, ib_cur)
            compute(b)
            issue_scatter(b, ib_cur)
            if do_idx:
                issue_idx(j + 4, ib_new)
            if do_fetch:
                wait_idx(j + 2, ib_wait)
                issue_fetch(j + 2, b, ib_wait)

        for jj in range(4):
            issue_idx(jj, jj)
        for jj in range(2):
            wait_idx(jj, jj)
            issue_fetch(jj, jj, jj)
        for jj in range(main_lo):
            step(jj, jj % _NBUF, jj % _NI, (jj + 2) % _NI, (jj + 4) % _NI,
                 do_ws=(jj >= 2))

        @pl.loop(main_lo, main_hi, step=6)
        def _(j0):
            for k in range(6):
                step(j0 + k, k % _NBUF, (main_lo + k) % _NI,
                     (main_lo + k + 2) % _NI, (main_lo + k + 4) % _NI)

        for j in range(main_hi, n_chunks):
            step(j, j % _NBUF, j % _NI, (j + 2) % _NI, (j + 4) % _NI,
                 do_idx=(j + 4 < n_chunks), do_fetch=(j + 2 < n_chunks))
        wait_scatter((n_chunks - 2) % _NBUF, (n_chunks - 2) % _NI)
        wait_scatter((n_chunks - 1) % _NBUF, (n_chunks - 1) % _NI)

        plsc.subcore_barrier()

        @pl.when(s < _NUM_SUBCORES - 1)
        def _():
            slab = pl.ds(s * slab_a, slab_a)
            pltpu.sync_copy(acc.at[slab, :], out_hbm.at[c, slab, :])

        @pl.when(s == _NUM_SUBCORES - 1)
        def _():
            slab = pl.ds(s * slab_a, slab_last)
            pltpu.sync_copy(acc.at[slab, :], out_hbm.at[c, slab, :])

    return sc_aggregate


def kernel(x, edge_index, edge_attr, W_same, b_same, W_diff, b_diff, W_G,
           W_last, b_last, u):
    N, F = x.shape
    E = edge_index.shape[1]
    K = edge_attr.shape[1]
    NB = 10
    BN = N // NB
    BE = 4000
    assert N % NB == 0 and E % BE == 0

    n_workers = _NUM_CORES * _NUM_SUBCORES
    idx3 = jnp.stack(
        (edge_index[0].reshape(n_workers, -1, _CHUNK),
         edge_index[1].reshape(n_workers, -1, _CHUNK)), axis=2)
    bd = b_diff.reshape(1, F)
    bs = b_same.reshape(1, F)
    bl = b_last.reshape(1, F)

    row_block = pl.BlockSpec((BN, F), lambda i: (i, 0))
    full_w = pl.BlockSpec((F, F), lambda i: (0, 0))
    full_b = pl.BlockSpec((1, F), lambda i: (0, 0))

    h, a2 = pl.pallas_call(
        _prelude_body,
        grid=(NB,),
        in_specs=[row_block, full_w, full_b, full_w, full_b],
        out_specs=[row_block, row_block],
        out_shape=[jax.ShapeDtypeStruct((N, F), jnp.float32)] * 2,
    )(x, W_diff, bd, W_same, bs)

    g = pl.pallas_call(
        _g_body,
        grid=(E // BE,),
        in_specs=[pl.BlockSpec((BE, K), lambda i: (i, 0)),
                  pl.BlockSpec((F, K), lambda i: (0, 0))],
        out_specs=pl.BlockSpec((BE, F), lambda i: (i, 0)),
        out_shape=jax.ShapeDtypeStruct((E, F), jnp.float32),
    )(edge_attr, W_G)

    parts = _make_sc_aggregate(N, F, E)(h, g, idx3, a2)

    out0, msged = pl.pallas_call(
        _tail_body,
        grid=(NB,),
        in_specs=[row_block, row_block, row_block, full_w, full_b, full_b],
        out_specs=[row_block, row_block],
        out_shape=[jax.ShapeDtypeStruct((N, F), jnp.float32)] * 2,
    )(parts[0], parts[1], x, W_last, bl, u)

    return out0, msged


# speedup vs baseline: 4.1375x; 1.0511x over previous
import functools

import jax
import jax.numpy as jnp
from jax import lax
from jax.experimental import pallas as pl
from jax.experimental.pallas import tpu as pltpu
from jax.experimental.pallas import tpu_sc as plsc

_PREC = lax.Precision.DEFAULT

_NUM_CORES = 2
_NUM_SUBCORES = 16
_LANES = 16
_CHUNK = 40
_NBUF = 2


def _ssp(v):
    return jax.nn.softplus(v) - jnp.log(2.0)


def _row_matmul(a, w):
    return lax.dot_general(a.astype(jnp.bfloat16), w.astype(jnp.bfloat16),
                           (((1,), (1,)), ((), ())),
                           preferred_element_type=jnp.float32,
                           precision=_PREC)


def _prelude_body(x_ref, wd_ref, bd_ref, ws_ref, bs_ref, h_ref, a2_ref):
    xa = _ssp(x_ref[...])
    h_ref[...] = _ssp(_row_matmul(xa, wd_ref[...]) + bd_ref[...])
    a2_ref[...] = 0.5 * _ssp(_row_matmul(xa, ws_ref[...]) + bs_ref[...])


def _g_body(ea_ref, wg_ref, g_ref):
    g_ref[...] = _row_matmul(ea_ref[...], wg_ref[...])


def _tail_body(p0_ref, p1_ref, x_ref, wl_ref, bl_ref, u_ref, out_ref, m_ref):
    m = p0_ref[...] + p1_ref[...]
    v = _row_matmul(_ssp(m), wl_ref[...]) + bl_ref[...]
    m_ref[...] = m
    out_ref[...] = v + x_ref[...] * u_ref[...]


_NI = 6


def _make_sc_aggregate(N, F, E):
    n_workers = _NUM_CORES * _NUM_SUBCORES
    epw = E // n_workers
    n_chunks = epw // _CHUNK
    assert epw % _CHUNK == 0
    main_lo, main_hi = 4, n_chunks - 6
    assert (main_hi - main_lo) % 6 == 0 and main_hi > main_lo
    slab_a = (N // _NUM_SUBCORES) // 8 * 8
    slab_last = N - slab_a * (_NUM_SUBCORES - 1)
    assert slab_a % 8 == 0 and slab_last % 8 == 0 and slab_last > 0

    mesh = plsc.VectorSubcoreMesh(core_axis_name="c", subcore_axis_name="s")

    @functools.partial(
        pl.kernel,
        mesh=mesh,
        out_type=jax.ShapeDtypeStruct((_NUM_CORES, N, F), jnp.float32),
        scratch_types=[
            pltpu.VMEM((_NI, 2, _CHUNK), jnp.int32),
            pltpu.VMEM((_NBUF, _CHUNK, F), jnp.float32),
            pltpu.VMEM((_NBUF, _CHUNK, F), jnp.float32),
            pltpu.VMEM((_NBUF, _CHUNK, F), jnp.float32),
            pltpu.VMEM_SHARED((N, F), jnp.float32),
            pltpu.SemaphoreType.DMA((_NI,)),
            pltpu.SemaphoreType.DMA((_NBUF,)),
            pltpu.SemaphoreType.DMA((_NBUF,)),
        ],
    )
    def sc_aggregate(h_hbm, g_hbm, src_hbm, dst_hbm, a2_hbm, out_hbm,
                     islot, rows, gbuf, mbuf, acc, isem, gsem, ssem):
        c = lax.axis_index("c")
        s = lax.axis_index("s")
        wid = c * _NUM_SUBCORES + s
        ebase = wid * epw

        @pl.when(s < _NUM_SUBCORES - 1)
        def _():
            slab = pl.ds(s * slab_a, slab_a)
            pltpu.sync_copy(a2_hbm.at[slab, :], acc.at[slab, :])

        @pl.when(s == _NUM_SUBCORES - 1)
        def _():
            slab = pl.ds(s * slab_a, slab_last)
            pltpu.sync_copy(a2_hbm.at[slab, :], acc.at[slab, :])

        plsc.subcore_barrier()

        def issue_idx(j, ib):
            pltpu.async_copy(src_hbm.at[wid, j], islot.at[ib, 0], isem.at[ib])
            pltpu.async_copy(dst_hbm.at[wid, j], islot.at[ib, 1], isem.at[ib])

        def wait_idx(j, ib):
            pltpu.make_async_copy(src_hbm.at[wid, j], islot.at[ib, 0],
                                  isem.at[ib]).wait()
            pltpu.make_async_copy(dst_hbm.at[wid, j], islot.at[ib, 1],
                                  isem.at[ib]).wait()

        def issue_fetch(j, b, ib):
            pltpu.async_copy(h_hbm.at[islot.at[ib, 0]], rows.at[b], gsem.at[b])
            pltpu.async_copy(g_hbm.at[pl.ds(ebase + j * _CHUNK, _CHUNK), :],
                             gbuf.at[b], gsem.at[b])

        def wait_fetch(j, b, ib):
            pltpu.make_async_copy(h_hbm.at[islot.at[ib, 0]], rows.at[b],
                                  gsem.at[b]).wait()
            pltpu.make_async_copy(g_hbm.at[pl.ds(ebase + j * _CHUNK, _CHUNK), :],
                                  gbuf.at[b], gsem.at[b]).wait()

        def issue_scatter(b, ib):
            pltpu.async_copy(mbuf.at[b], acc.at[islot.at[ib, 1]], ssem.at[b],
                             add=True)

        def wait_scatter(b, ib):
            pltpu.make_async_copy(mbuf.at[b], acc.at[islot.at[ib, 1]],
                                  ssem.at[b]).wait()

        def compute(b):
            @pl.loop(0, _CHUNK)
            def _(i):
                for f in range(0, F, _LANES):
                    sl = pl.ds(f, _LANES)
                    mbuf[b, i, sl] = rows[b, i, sl] * gbuf[b, i, sl]

        def step(j, b, ib_cur, ib_wait, ib_new,
                 do_ws=True, do_idx=True, do_fetch=True):
            if do_ws:
                wait_scatter(b, ib_cur)
            wait_fetch(j, b, ib_cur)
            compute(b)
            issue_scatter(b, ib_cur)
            if do_idx:
                issue_idx(j + 4, ib_new)
            if do_fetch:
                wait_idx(j + 2, ib_wait)
                issue_fetch(j + 2, b, ib_wait)

        for jj in range(4):
            issue_idx(jj, jj)
        for jj in range(2):
            wait_idx(jj, jj)
            issue_fetch(jj, jj, jj)
        for jj in range(main_lo):
            step(jj, jj % _NBUF, jj % _NI, (jj + 2) % _NI, (jj + 4) % _NI,
                 do_ws=(jj >= 2))

        @pl.loop(main_lo, main_hi, step=6)
        def _(j0):
            for k in range(6):
                step(j0 + k, k % _NBUF, (main_lo + k) % _NI,
                     (main_lo + k + 2) % _NI, (main_lo + k + 4) % _NI)

        for j in range(main_hi, n_chunks):
            step(j, j % _NBUF, j % _NI, (j + 2) % _NI, (j + 4) % _NI,
                 do_idx=(j + 4 < n_chunks), do_fetch=(j + 2 < n_chunks))
        wait_scatter((n_chunks - 2) % _NBUF, (n_chunks - 2) % _NI)
        wait_scatter((n_chunks - 1) % _NBUF, (n_chunks - 1) % _NI)

        plsc.subcore_barrier()

        @pl.when(s < _NUM_SUBCORES - 1)
        def _():
            slab = pl.ds(s * slab_a, slab_a)
            pltpu.sync_copy(acc.at[slab, :], out_hbm.at[c, slab, :])

        @pl.when(s == _NUM_SUBCORES - 1)
        def _():
            slab = pl.ds(s * slab_a, slab_last)
            pltpu.sync_copy(acc.at[slab, :], out_hbm.at[c, slab, :])

    return sc_aggregate


def kernel(x, edge_index, edge_attr, W_same, b_same, W_diff, b_diff, W_G,
           W_last, b_last, u):
    N, F = x.shape
    E = edge_index.shape[1]
    K = edge_attr.shape[1]
    NB = 10
    BN = N // NB
    BE = 4000
    assert N % NB == 0 and E % BE == 0

    n_workers = _NUM_CORES * _NUM_SUBCORES
    src3 = edge_index[0].reshape(n_workers, -1, _CHUNK)
    dst3 = edge_index[1].reshape(n_workers, -1, _CHUNK)
    bd = b_diff.reshape(1, F)
    bs = b_same.reshape(1, F)
    bl = b_last.reshape(1, F)

    row_block = pl.BlockSpec((BN, F), lambda i: (i, 0))
    full_w = pl.BlockSpec((F, F), lambda i: (0, 0))
    full_b = pl.BlockSpec((1, F), lambda i: (0, 0))

    h, a2 = pl.pallas_call(
        _prelude_body,
        grid=(NB,),
        in_specs=[row_block, full_w, full_b, full_w, full_b],
        out_specs=[row_block, row_block],
        out_shape=[jax.ShapeDtypeStruct((N, F), jnp.float32)] * 2,
    )(x, W_diff, bd, W_same, bs)

    g = pl.pallas_call(
        _g_body,
        grid=(E // BE,),
        in_specs=[pl.BlockSpec((BE, K), lambda i: (i, 0)),
                  pl.BlockSpec((F, K), lambda i: (0, 0))],
        out_specs=pl.BlockSpec((BE, F), lambda i: (i, 0)),
        out_shape=jax.ShapeDtypeStruct((E, F), jnp.float32),
    )(edge_attr, W_G)

    parts = _make_sc_aggregate(N, F, E)(h, g, src3, dst3, a2)

    out0, msged = pl.pallas_call(
        _tail_body,
        grid=(NB,),
        in_specs=[row_block, row_block, row_block, full_w, full_b, full_b],
        out_specs=[row_block, row_block],
        out_shape=[jax.ShapeDtypeStruct((N, F), jnp.float32)] * 2,
    )(parts[0], parts[1], x, W_last, bl, u)

    return out0, msged
